# Initial kernel scaffold; baseline (speedup 1.0000x reference)
import jax
import jax.numpy as jnp
from jax.experimental import pallas as pl


def kernel(queries, ling, visual, filter_idx, ent_emb, rel_emb, W_ling, W_visual):
    raise NotImplementedError("write your pallas kernel here")


# speedup vs baseline: 1.0536x; 1.0536x over previous
import functools

import jax
import jax.numpy as jnp
from jax import lax
from jax.experimental import pallas as pl
from jax.experimental.pallas import tpu as pltpu
from jax.experimental.pallas import tpu_sc as plsc

NEG = -1000000.0
NUM_WORKERS = 32


def _gather_rows(ht, r, ent_emb, ling, visual, rel_emb, *, interpret=False):
    nht_total = ht.shape[0]
    nr_total = r.shape[0]
    d = ent_emb.shape[1]
    nht = nht_total // NUM_WORKERS
    nr = nr_total // NUM_WORKERS
    mesh = plsc.VectorSubcoreMesh(core_axis_name="c", subcore_axis_name="s")

    @functools.partial(
        pl.kernel,
        out_type=[jax.ShapeDtypeStruct((nht_total, d), jnp.float32)] * 3
        + [jax.ShapeDtypeStruct((nr_total, d), jnp.float32)],
        mesh=mesh,
        scratch_types=[
            pltpu.VMEM((nht,), jnp.int32),
            pltpu.VMEM((nr,), jnp.int32),
            pltpu.VMEM((nht, d), jnp.float32),
            pltpu.VMEM((nr, d), jnp.float32),
            pltpu.SemaphoreType.DMA,
        ],
        interpret=interpret,
    )
    def k(ht_hbm, r_hbm, ent_hbm, ling_hbm, vis_hbm, rel_hbm,
          oe, ol, ov, orel, idxh, idxr, bufh, bufr, sem):
        wid = lax.axis_index("s") * 2 + lax.axis_index("c")
        bh = wid * nht
        br = wid * nr
        pltpu.sync_copy(ht_hbm.at[pl.ds(bh, nht)], idxh)
        pltpu.sync_copy(r_hbm.at[pl.ds(br, nr)], idxr)
        for table, out in ((ent_hbm, oe), (ling_hbm, ol), (vis_hbm, ov)):
            pltpu.async_copy(table.at[idxh], bufh, sem).wait()
            pltpu.sync_copy(bufh, out.at[pl.ds(bh, nht)])
        pltpu.async_copy(rel_hbm.at[idxr], bufr, sem).wait()
        pltpu.sync_copy(bufr, orel.at[pl.ds(br, nr)])

    return k(ht, r, ent_emb, ling, visual, rel_emb)


def _prep_body(er, lr, vr, rr, wl, wv, q_ref, tgt_ref):
    b = rr.shape[0]
    fused = (
        er[...]
        + jnp.dot(lr[...], wl[...], preferred_element_type=jnp.float32)
        + jnp.dot(vr[...], wv[...], preferred_element_type=jnp.float32)
    )
    q = fused[:b, :] * rr[...]
    q_ref[...] = q
    tgt_ref[...] = jnp.sum(q * fused[b:, :], axis=1, keepdims=True)


def _prep_call(er, lr, vr, rr, wl, wv, *, interpret=False):
    b, d = rr.shape
    return pl.pallas_call(
        _prep_body,
        out_shape=[
            jax.ShapeDtypeStruct((b, d), jnp.float32),
            jax.ShapeDtypeStruct((b, 1), jnp.float32),
        ],
        interpret=interpret,
    )(er, lr, vr, rr, wl, wv)


def _main_body(c_tile, n, q, tgt, ent, lingb, visb, wl, wv, scores_ref, cnt_ref):
    j = pl.program_id(0)
    ae = (
        ent[...]
        + jnp.dot(lingb[...], wl[...], preferred_element_type=jnp.float32)
        + jnp.dot(visb[...], wv[...], preferred_element_type=jnp.float32)
    )
    s = lax.dot_general(q[...], ae, (((1,), (1,)), ((), ())),
                        preferred_element_type=jnp.float32)
    scores_ref[...] = s
    col = j * c_tile + lax.broadcasted_iota(jnp.int32, s.shape, 1)
    ge = jnp.where((s >= tgt[...]) & (col < n), 1.0, 0.0)

    @pl.when(j == 0)
    def _():
        cnt_ref[...] = jnp.zeros_like(cnt_ref)

    cnt_ref[...] += jnp.sum(ge, axis=1, keepdims=True)


def _main_call(q, tgt, ent, ling, vis, wl, wv, *, c_tile=1024, interpret=False):
    b, d = q.shape
    n = ent.shape[0]
    grid = (pl.cdiv(n, c_tile),)
    return pl.pallas_call(
        functools.partial(_main_body, c_tile, n),
        grid=grid,
        in_specs=[
            pl.BlockSpec((b, d), lambda j: (0, 0)),
            pl.BlockSpec((b, 1), lambda j: (0, 0)),
            pl.BlockSpec((c_tile, d), lambda j: (j, 0)),
            pl.BlockSpec((c_tile, d), lambda j: (j, 0)),
            pl.BlockSpec((c_tile, d), lambda j: (j, 0)),
            pl.BlockSpec((d, d), lambda j: (0, 0)),
            pl.BlockSpec((d, d), lambda j: (0, 0)),
        ],
        out_specs=[
            pl.BlockSpec((b, c_tile), lambda j: (0, j)),
            pl.BlockSpec((b, 1), lambda j: (0, 0)),
        ],
        out_shape=[
            jax.ShapeDtypeStruct((b, n), jnp.float32),
            jax.ShapeDtypeStruct((b, 1), jnp.float32),
        ],
        interpret=interpret,
    )(q, tgt, ent, ling, vis, wl, wv)


def _scatter_call(fidx3, marker3, neg2, scores_flat_ref, *, interpret=False):
    nw, ch, lw = fidx3.shape
    mesh = plsc.VectorSubcoreMesh(core_axis_name="c", subcore_axis_name="s")

    @functools.partial(
        pl.kernel,
        out_type=[jax.ShapeDtypeStruct((nw, ch, lw), jnp.float32)] * 2,
        mesh=mesh,
        scratch_types=[
            pltpu.VMEM((ch, lw), jnp.int32),
            pltpu.VMEM((ch, lw), jnp.float32),
            pltpu.VMEM((ch, lw), jnp.float32),
            pltpu.VMEM((ch, lw), jnp.float32),
            pltpu.VMEM((ch, lw), jnp.float32),
            pltpu.SemaphoreType.DMA,
        ],
        interpret=interpret,
    )
    def k(fidx_hbm, mk_hbm, neg_hbm, scores_hbm, vals_out, g2_out,
          idxv, valv, g2v, mkv, negv, sem):
        wid = lax.axis_index("s") * 2 + lax.axis_index("c")
        pltpu.sync_copy(fidx_hbm.at[wid], idxv)
        pltpu.sync_copy(mk_hbm.at[wid], mkv)
        pltpu.sync_copy(neg_hbm, negv)
        ds = [pltpu.async_copy(scores_hbm.at[idxv.at[c]], valv.at[c], sem)
              for c in range(ch)]
        for dd in ds:
            dd.wait()
        pltpu.sync_copy(valv, vals_out.at[wid])
        ds = [pltpu.async_copy(mkv.at[c], scores_hbm.at[idxv.at[c]], sem)
              for c in range(ch)]
        for dd in ds:
            dd.wait()
        ds = [pltpu.async_copy(scores_hbm.at[idxv.at[c]], g2v.at[c], sem)
              for c in range(ch)]
        for dd in ds:
            dd.wait()
        pltpu.sync_copy(g2v, g2_out.at[wid])
        ds = [pltpu.async_copy(negv.at[c], scores_hbm.at[idxv.at[c]], sem)
              for c in range(ch)]
        for dd in ds:
            dd.wait()

    return k(fidx3, marker3, neg2, scores_flat_ref)


def _finalize_body(vals, g2, tgt, cnt, ranks_ref):
    b, kk = vals.shape
    m = (lax.broadcasted_iota(jnp.int32, (b, kk), 0) * kk
         + lax.broadcasted_iota(jnp.int32, (b, kk), 1)).astype(jnp.float32)
    winner = g2[...] == m
    cmp = jnp.where(vals[...] >= tgt[...], 1.0, 0.0)
    negcmp = jnp.where(NEG >= tgt[...], 1.0, 0.0)
    corr = jnp.sum(jnp.where(winner, cmp - negcmp, 0.0), axis=1, keepdims=True)
    ranks_ref[...] = 1.0 + cnt[...] - corr


def _finalize_call(vals, g2, tgt, cnt, *, interpret=False):
    b = vals.shape[0]
    return pl.pallas_call(
        _finalize_body,
        out_shape=jax.ShapeDtypeStruct((b, 1), jnp.float32),
        interpret=interpret,
    )(vals, g2, tgt, cnt)


def kernel(queries, ling, visual, filter_idx, ent_emb, rel_emb, W_ling, W_visual):
    b = queries.shape[0]
    n, d = ent_emb.shape
    f = filter_idx.shape[1]
    h = queries[:, 0].astype(jnp.int32)
    r = queries[:, 1].astype(jnp.int32)
    t = queries[:, 2].astype(jnp.int32)
    ht = jnp.concatenate([h, t])

    er, lr, vr, rr = _gather_rows(ht, r, ent_emb, ling, visual, rel_emb)
    q, tgt = _prep_call(er, lr, vr, rr, W_ling, W_visual)
    scores, cnt = _main_call(q, tgt, ent_emb, ling, visual, W_ling, W_visual)

    kk = f + 2
    idx_all = jnp.concatenate(
        [filter_idx.astype(jnp.int32), t[:, None], t[:, None]], axis=1)
    fidx = jnp.arange(b, dtype=jnp.int32)[:, None] * n + idx_all
    e_total = b * kk
    ch = e_total // (NUM_WORKERS * 128)
    fidx3 = fidx.reshape(NUM_WORKERS, ch, 128)
    marker3 = jnp.arange(e_total, dtype=jnp.float32).reshape(NUM_WORKERS, ch, 128)
    neg2 = jnp.full((ch, 128), NEG, jnp.float32)

    sref = jax.new_ref(scores.reshape(-1))
    vals3, g23 = _scatter_call(fidx3, marker3, neg2, sref)
    filtered = sref[...].reshape(b, n)

    ranks2 = _finalize_call(vals3.reshape(b, kk), g23.reshape(b, kk), tgt, cnt)
    return filtered, tgt, ranks2.reshape(b)


# speedup vs baseline: 2.1296x; 2.0212x over previous
import functools

import jax
import jax.numpy as jnp
from jax import lax
from jax.experimental import pallas as pl
from jax.experimental.pallas import tpu as pltpu
from jax.experimental.pallas import tpu_sc as plsc

NEG = -1000000.0
NUM_WORKERS = 32


def _gather_rows(ht, r, ent_emb, ling, visual, rel_emb, *, interpret=False):
    nht_total = ht.shape[0]
    nr_total = r.shape[0]
    d = ent_emb.shape[1]
    nht = nht_total // NUM_WORKERS
    nr = nr_total // NUM_WORKERS
    mesh = plsc.VectorSubcoreMesh(core_axis_name="c", subcore_axis_name="s")

    @functools.partial(
        pl.kernel,
        out_type=[jax.ShapeDtypeStruct((nht_total, d), jnp.float32)] * 3
        + [jax.ShapeDtypeStruct((nr_total, d), jnp.float32)],
        mesh=mesh,
        scratch_types=[
            pltpu.VMEM((nht,), jnp.int32),
            pltpu.VMEM((nr,), jnp.int32),
            pltpu.VMEM((nht, d), jnp.float32),
            pltpu.VMEM((nr, d), jnp.float32),
            pltpu.SemaphoreType.DMA,
        ],
        interpret=interpret,
    )
    def k(ht_hbm, r_hbm, ent_hbm, ling_hbm, vis_hbm, rel_hbm,
          oe, ol, ov, orel, idxh, idxr, bufh, bufr, sem):
        wid = lax.axis_index("s") * 2 + lax.axis_index("c")
        bh = wid * nht
        br = wid * nr
        pltpu.sync_copy(ht_hbm.at[pl.ds(bh, nht)], idxh)
        pltpu.sync_copy(r_hbm.at[pl.ds(br, nr)], idxr)
        for table, out in ((ent_hbm, oe), (ling_hbm, ol), (vis_hbm, ov)):
            pltpu.async_copy(table.at[idxh], bufh, sem).wait()
            pltpu.sync_copy(bufh, out.at[pl.ds(bh, nht)])
        pltpu.async_copy(rel_hbm.at[idxr], bufr, sem).wait()
        pltpu.sync_copy(bufr, orel.at[pl.ds(br, nr)])

    return k(ht, r, ent_emb, ling, visual, rel_emb)


def _prep_body(er, lr, vr, rr, wl, wv, q_ref, tgt_ref):
    b = rr.shape[0]
    fused = (
        er[...]
        + jnp.dot(lr[...], wl[...], preferred_element_type=jnp.float32)
        + jnp.dot(vr[...], wv[...], preferred_element_type=jnp.float32)
    )
    q = fused[:b, :] * rr[...]
    q_ref[...] = q
    tgt_ref[...] = jnp.sum(q * fused[b:, :], axis=1, keepdims=True)


def _prep_call(er, lr, vr, rr, wl, wv, *, interpret=False):
    b, d = rr.shape
    return pl.pallas_call(
        _prep_body,
        out_shape=[
            jax.ShapeDtypeStruct((b, d), jnp.float32),
            jax.ShapeDtypeStruct((b, 1), jnp.float32),
        ],
        interpret=interpret,
    )(er, lr, vr, rr, wl, wv)


def _main_body(c_tile, n, q, tgt, ent, lingb, visb, wl, wv, scores_ref, cnt_ref):
    j = pl.program_id(0)
    ae = (
        ent[...]
        + jnp.dot(lingb[...], wl[...], preferred_element_type=jnp.float32)
        + jnp.dot(visb[...], wv[...], preferred_element_type=jnp.float32)
    )
    s = lax.dot_general(q[...], ae, (((1,), (1,)), ((), ())),
                        preferred_element_type=jnp.float32)
    scores_ref[...] = s
    col = j * c_tile + lax.broadcasted_iota(jnp.int32, s.shape, 1)
    ge = jnp.where((s >= tgt[...]) & (col < n), 1.0, 0.0)

    @pl.when(j == 0)
    def _():
        cnt_ref[...] = jnp.zeros_like(cnt_ref)

    cnt_ref[...] += jnp.sum(ge, axis=1, keepdims=True)


def _main_call(q, tgt, ent, ling, vis, wl, wv, *, c_tile=1024, interpret=False):
    b, d = q.shape
    n = ent.shape[0]
    grid = (pl.cdiv(n, c_tile),)
    return pl.pallas_call(
        functools.partial(_main_body, c_tile, n),
        grid=grid,
        in_specs=[
            pl.BlockSpec((b, d), lambda j: (0, 0)),
            pl.BlockSpec((b, 1), lambda j: (0, 0)),
            pl.BlockSpec((c_tile, d), lambda j: (j, 0)),
            pl.BlockSpec((c_tile, d), lambda j: (j, 0)),
            pl.BlockSpec((c_tile, d), lambda j: (j, 0)),
            pl.BlockSpec((d, d), lambda j: (0, 0)),
            pl.BlockSpec((d, d), lambda j: (0, 0)),
        ],
        out_specs=[
            pl.BlockSpec((b, c_tile), lambda j: (0, j)),
            pl.BlockSpec((b, 1), lambda j: (0, 0)),
        ],
        out_shape=[
            jax.ShapeDtypeStruct((b, n), jnp.float32),
            jax.ShapeDtypeStruct((b, 1), jnp.float32),
        ],
        interpret=interpret,
    )(q, tgt, ent, ling, vis, wl, wv)


def _filter_copy(scores, idx64, *, interpret=False):
    b, n = scores.shape
    kk = idx64.shape[1]
    nrows = b // NUM_WORKERS
    ngrp = kk // 16
    mesh = plsc.VectorSubcoreMesh(core_axis_name="c", subcore_axis_name="s")

    @functools.partial(
        pl.kernel,
        out_type=[
            jax.ShapeDtypeStruct((b, n), jnp.float32),
            jax.ShapeDtypeStruct((b, kk), jnp.float32),
            jax.ShapeDtypeStruct((b, kk), jnp.float32),
        ],
        mesh=mesh,
        scratch_types=[
            pltpu.VMEM((n,), jnp.float32),
            pltpu.VMEM((n,), jnp.float32),
            pltpu.VMEM((kk,), jnp.int32),
            pltpu.VMEM((kk,), jnp.float32),
            pltpu.VMEM((kk,), jnp.float32),
            pltpu.SemaphoreType.DMA,
            pltpu.SemaphoreType.DMA,
        ],
        compiler_params=pltpu.CompilerParams(needs_layout_passes=False),
        interpret=interpret,
    )
    def k(scores_hbm, idx_hbm, filt_hbm, vals_hbm, g2_hbm,
          row0, row1, idxv, valv, g2v, insem, outsem):
        wid = lax.axis_index("s") * 2 + lax.axis_index("c")
        base = wid * nrows
        bufs = (row0, row1)
        out_dmas = [None, None]
        in_dma = pltpu.async_copy(scores_hbm.at[base], row0, insem)
        for step in range(nrows):
            slot = step % 2
            row = base + step
            rb = bufs[slot]
            in_dma.wait()
            if step + 1 < nrows:
                nslot = 1 - slot
                if out_dmas[nslot] is not None:
                    out_dmas[nslot].wait()
                    out_dmas[nslot] = None
                in_dma = pltpu.async_copy(
                    scores_hbm.at[row + 1], bufs[nslot], insem)
            pltpu.sync_copy(idx_hbm.at[row], idxv)
            for g in range(ngrp):
                sl = pl.ds(g * 16, 16)
                valv[sl] = plsc.load_gather(rb, [idxv[sl]])
            for g in range(ngrp):
                sl = pl.ds(g * 16, 16)
                mk = (lax.iota(jnp.int32, 16) + g * 16).astype(jnp.float32)
                plsc.store_scatter(rb, [idxv[sl]], mk)
            for g in range(ngrp):
                sl = pl.ds(g * 16, 16)
                g2v[sl] = plsc.load_gather(rb, [idxv[sl]])
            for g in range(ngrp):
                sl = pl.ds(g * 16, 16)
                plsc.store_scatter(rb, [idxv[sl]],
                                   jnp.full((16,), NEG, jnp.float32))
            pltpu.sync_copy(valv, vals_hbm.at[row])
            pltpu.sync_copy(g2v, g2_hbm.at[row])
            out_dmas[slot] = pltpu.async_copy(rb, filt_hbm.at[row], outsem)
        for od in out_dmas:
            if od is not None:
                od.wait()

    return k(scores, idx64)


def _finalize_body(vals, g2, tgt, cnt, ranks_ref):
    b, kk = vals.shape
    m = lax.broadcasted_iota(jnp.int32, (b, kk), 1).astype(jnp.float32)
    winner = g2[...] == m
    cmp = jnp.where(vals[...] >= tgt[...], 1.0, 0.0)
    negcmp = jnp.where(NEG >= tgt[...], 1.0, 0.0)
    corr = jnp.sum(jnp.where(winner, cmp - negcmp, 0.0), axis=1, keepdims=True)
    ranks_ref[...] = 1.0 + cnt[...] - corr


def _finalize_call(vals, g2, tgt, cnt, *, interpret=False):
    b = vals.shape[0]
    return pl.pallas_call(
        _finalize_body,
        out_shape=jax.ShapeDtypeStruct((b, 1), jnp.float32),
        interpret=interpret,
    )(vals, g2, tgt, cnt)


def kernel(queries, ling, visual, filter_idx, ent_emb, rel_emb, W_ling, W_visual):
    b = queries.shape[0]
    n, d = ent_emb.shape
    f = filter_idx.shape[1]
    h = queries[:, 0].astype(jnp.int32)
    r = queries[:, 1].astype(jnp.int32)
    t = queries[:, 2].astype(jnp.int32)
    ht = jnp.concatenate([h, t])

    er, lr, vr, rr = _gather_rows(ht, r, ent_emb, ling, visual, rel_emb)
    q, tgt = _prep_call(er, lr, vr, rr, W_ling, W_visual)
    scores, cnt = _main_call(q, tgt, ent_emb, ling, visual, W_ling, W_visual)

    kk = -(-(f + 1) // 16) * 16
    idx64 = jnp.concatenate(
        [filter_idx.astype(jnp.int32),
         jnp.broadcast_to(t[:, None], (b, kk - f))], axis=1)

    filtered, vals, g2 = _filter_copy(scores, idx64)
    ranks2 = _finalize_call(vals, g2, tgt, cnt)
    return filtered, tgt, ranks2.reshape(b)
